# baseline (speedup 1.0000x reference)
import jax
import jax.numpy as jnp
from jax import lax
from jax.experimental import pallas as pl
from jax.experimental.pallas import tpu as pltpu
from jax.experimental.pallas import tpu_sc as plsc

N = 10000
E = 320000
D = 128
H = 4 * D

NC = 2
NS = 16
HD = D // NC
EPT = E // NS
K = 40
NCHUNK = EPT // K
NB = 5
NPAD = 10240
RPT = NPAD // NS
DEGW = 16


def _sc_body(zs_hbm, src_hbm, dst_hbm, zagg_hbm, zdeg_hbm, ones_hbm,
             agg_out, deg_out, src_v, dst_v, rows_v, ones_v,
             agg_sh, deg_sh, sem_g, sem_s, sem_d):
    cid = lax.axis_index("c")
    sid = lax.axis_index("s")
    r0 = sid * RPT
    stage = [
        pltpu.async_copy(src_hbm.at[sid], src_v, sem_g),
        pltpu.async_copy(dst_hbm.at[sid], dst_v, sem_g),
        pltpu.async_copy(ones_hbm, ones_v, sem_g),
        pltpu.async_copy(zagg_hbm.at[pl.ds(r0, RPT)],
                         agg_sh.at[pl.ds(r0, RPT)], sem_s),
        pltpu.async_copy(zdeg_hbm.at[pl.ds(r0, RPT)],
                         deg_sh.at[pl.ds(r0, RPT)], sem_s),
    ]
    for cp in stage:
        cp.wait()
    plsc.subcore_barrier()

    def gather(j0, bs):
        return [
            pltpu.async_copy(zs_hbm.at[cid].at[src_v.at[j0 + b]],
                             rows_v.at[bs + b], sem_g)
            for b in range(NB)
        ]

    def scatter(j0, bs):
        return [
            pltpu.async_copy(rows_v.at[bs + b], agg_sh.at[dst_v.at[j0 + b]],
                             sem_s, add=True)
            for b in range(NB)
        ]

    def deg_scatter(j0):
        dcps = [
            pltpu.async_copy(ones_v, deg_sh.at[dst_v.at[j0 + b]],
                             sem_d, add=True)
            for b in range(NB)
        ]
        for cp in dcps:
            cp.wait()

    @pl.loop(0, NCHUNK // (2 * NB))
    def _(jo):
        j0 = jo * 2 * NB
        ga = gather(j0, 0)
        for cp in ga:
            cp.wait()
        sa = scatter(j0, 0)
        gb = gather(j0 + NB, NB)

        @pl.when(cid == 0)
        def _():
            deg_scatter(j0)

        for cp in sa:
            cp.wait()
        for cp in gb:
            cp.wait()
        sb = scatter(j0 + NB, NB)

        @pl.when(cid == 1)
        def _():
            deg_scatter(j0 + NB)

        for cp in sb:
            cp.wait()

    plsc.subcore_barrier()
    pltpu.sync_copy(agg_sh.at[pl.ds(r0, RPT)], agg_out.at[cid, pl.ds(r0, RPT)])
    pltpu.sync_copy(deg_sh.at[pl.ds(r0, RPT)], deg_out.at[cid, pl.ds(r0, RPT)])


_sc_aggregate = pl.kernel(
    _sc_body,
    out_type=(
        jax.ShapeDtypeStruct((NC, NPAD, HD), jnp.float32),
        jax.ShapeDtypeStruct((NC, NPAD, DEGW), jnp.float32),
    ),
    mesh=plsc.VectorSubcoreMesh(
        core_axis_name="c", subcore_axis_name="s",
        num_cores=NC, num_subcores=NS),
    scratch_types=(
        pltpu.VMEM((NCHUNK, K), jnp.int32),
        pltpu.VMEM((NCHUNK, K), jnp.int32),
        pltpu.VMEM((2 * NB, K, HD), jnp.float32),
        pltpu.VMEM((K, DEGW), jnp.float32),
        pltpu.VMEM_SHARED((NPAD, HD), jnp.float32),
        pltpu.VMEM_SHARED((NPAD, DEGW), jnp.float32),
        pltpu.SemaphoreType.DMA,
        pltpu.SemaphoreType.DMA,
        pltpu.SemaphoreType.DMA,
    ),
    compiler_params=pltpu.CompilerParams(use_tc_tiling_on_sc=False),
)


def _tc_body(z_ref, agg_ref, deg_ref, ws_ref, wn_ref, bg_ref,
             w1_ref, b1_ref, w2_ref, b2_ref, w3_ref, b3_ref, out_ref):
    agg = jnp.concatenate([agg_ref[0], agg_ref[1]], axis=-1)
    deg = deg_ref[0, :, 0:1] + deg_ref[1, :, 0:1]
    mean = agg / jnp.maximum(deg, 1.0)
    f32 = jnp.float32
    h = (jnp.dot(z_ref[...], ws_ref[...], preferred_element_type=f32)
         + jnp.dot(mean, wn_ref[...], preferred_element_type=f32)
         + bg_ref[...])
    h = jnp.tanh(jnp.dot(h, w1_ref[...], preferred_element_type=f32)
                 + b1_ref[...])
    h = jnp.tanh(jnp.dot(h, w2_ref[...], preferred_element_type=f32)
                 + b2_ref[...])
    out_ref[...] = (jnp.dot(h, w3_ref[...], preferred_element_type=f32)
                    + b3_ref[...])


BR = 1000


def _tc_mlp(z, agg, deg, W_self, W_neigh, b_gnn, W1, b1, W2, b2, W3, b3):
    full = lambda s: pl.BlockSpec(s, lambda i: (0,) * len(s))
    return pl.pallas_call(
        _tc_body,
        grid=(N // BR,),
        in_specs=[
            pl.BlockSpec((BR, D), lambda i: (i, 0)),
            pl.BlockSpec((NC, BR, HD), lambda i: (0, i, 0)),
            pl.BlockSpec((NC, BR, DEGW), lambda i: (0, i, 0)),
            full((D, D)), full((D, D)), full((1, D)),
            full((D, H)), full((1, H)),
            full((H, H)), full((1, H)),
            full((H, D)), full((1, D)),
        ],
        out_specs=pl.BlockSpec((BR, D), lambda i: (i, 0)),
        out_shape=jax.ShapeDtypeStruct((N, D), jnp.float32),
    )(z, agg, deg, W_self, W_neigh, b_gnn.reshape(1, D),
      W1, b1.reshape(1, H), W2, b2.reshape(1, H), W3, b3.reshape(1, D))


def kernel(t, z, edge_index, W_self, W_neigh, b_gnn, W1, b1, W2, b2, W3, b3):
    zs = jnp.stack([z[:, :HD], z[:, HD:]])
    src = edge_index[0].reshape(NS, NCHUNK, K)
    dst = edge_index[1].reshape(NS, NCHUNK, K)
    zagg = jnp.zeros((NPAD, HD), jnp.float32)
    zdeg = jnp.zeros((NPAD, DEGW), jnp.float32)
    ones = jnp.ones((K, DEGW), jnp.float32)
    agg, deg = _sc_aggregate(zs, src, dst, zagg, zdeg, ones)
    return _tc_mlp(z, agg, deg, W_self, W_neigh, b_gnn,
                   W1, b1, W2, b2, W3, b3)


# speedup vs baseline: 1.0896x; 1.0896x over previous
import jax
import jax.numpy as jnp
from jax import lax
from jax.experimental import pallas as pl
from jax.experimental.pallas import tpu as pltpu
from jax.experimental.pallas import tpu_sc as plsc

N = 10000
E = 320000
D = 128
H = 4 * D

NC = 2
NS = 16
NW = NC * NS
EPW = E // NW
K = 40
NCK = EPW // K
NB = 5
NPAD = 10240
RPT = NPAD // NS
DEGW = 16


def _sc_agg_body(z_hbm, src_hbm, dst_hbm, zagg_hbm,
                 agg_out, src_v, dst_v, rows_v, agg_sh, sem_g, sem_s):
    cid = lax.axis_index("c")
    sid = lax.axis_index("s")
    wid = cid * NS + sid
    r0 = sid * RPT
    stage = [
        pltpu.async_copy(src_hbm.at[wid], src_v, sem_g),
        pltpu.async_copy(dst_hbm.at[wid], dst_v, sem_g),
        pltpu.async_copy(zagg_hbm.at[pl.ds(r0, RPT)],
                         agg_sh.at[pl.ds(r0, RPT)], sem_s),
    ]
    for cp in stage:
        cp.wait()
    plsc.subcore_barrier()

    @pl.loop(0, NCK // NB)
    def _(jo):
        j0 = jo * NB
        gcps = [
            pltpu.async_copy(z_hbm.at[src_v.at[j0 + b]], rows_v.at[b], sem_g)
            for b in range(NB)
        ]
        for cp in gcps:
            cp.wait()
        scps = [
            pltpu.async_copy(rows_v.at[b], agg_sh.at[dst_v.at[j0 + b]],
                             sem_s, add=True)
            for b in range(NB)
        ]
        for cp in scps:
            cp.wait()

    plsc.subcore_barrier()
    pltpu.sync_copy(agg_sh.at[pl.ds(r0, RPT)], agg_out.at[cid, pl.ds(r0, RPT)])


_sc_aggregate = pl.kernel(
    _sc_agg_body,
    out_type=jax.ShapeDtypeStruct((NC, NPAD, D), jnp.float32),
    mesh=plsc.VectorSubcoreMesh(
        core_axis_name="c", subcore_axis_name="s",
        num_cores=NC, num_subcores=NS),
    scratch_types=(
        pltpu.VMEM((NCK, K), jnp.int32),
        pltpu.VMEM((NCK, K), jnp.int32),
        pltpu.VMEM((NB, K, D), jnp.float32),
        pltpu.VMEM_SHARED((NPAD, D), jnp.float32),
        pltpu.SemaphoreType.DMA,
        pltpu.SemaphoreType.DMA,
    ),
    compiler_params=pltpu.CompilerParams(use_tc_tiling_on_sc=False),
)


def _sc_deg_body(dst_hbm, zdeg_hbm, ones_hbm,
                 deg_out, dst_v, ones_v, deg_sh, sem_d, sem_s):
    cid = lax.axis_index("c")
    sid = lax.axis_index("s")
    wid = cid * NS + sid
    r0 = sid * RPT
    stage = [
        pltpu.async_copy(dst_hbm.at[wid], dst_v, sem_d),
        pltpu.async_copy(ones_hbm, ones_v, sem_d),
        pltpu.async_copy(zdeg_hbm.at[pl.ds(r0, RPT)],
                         deg_sh.at[pl.ds(r0, RPT)], sem_s),
    ]
    for cp in stage:
        cp.wait()
    plsc.subcore_barrier()

    @pl.loop(0, NCK // NB)
    def _(jo):
        j0 = jo * NB
        dcps = [
            pltpu.async_copy(ones_v, deg_sh.at[dst_v.at[j0 + b]],
                             sem_s, add=True)
            for b in range(NB)
        ]
        for cp in dcps:
            cp.wait()

    plsc.subcore_barrier()
    pltpu.sync_copy(deg_sh.at[pl.ds(r0, RPT)], deg_out.at[cid, pl.ds(r0, RPT)])


_sc_degree = pl.kernel(
    _sc_deg_body,
    out_type=jax.ShapeDtypeStruct((NC, NPAD, DEGW), jnp.float32),
    mesh=plsc.VectorSubcoreMesh(
        core_axis_name="c", subcore_axis_name="s",
        num_cores=NC, num_subcores=NS),
    scratch_types=(
        pltpu.VMEM((NCK, K), jnp.int32),
        pltpu.VMEM((K, DEGW), jnp.float32),
        pltpu.VMEM_SHARED((NPAD, DEGW), jnp.float32),
        pltpu.SemaphoreType.DMA,
        pltpu.SemaphoreType.DMA,
    ),
    compiler_params=pltpu.CompilerParams(use_tc_tiling_on_sc=False),
)


def _tc_body(z_ref, agg_ref, deg_ref, ws_ref, wn_ref, bg_ref,
             w1_ref, b1_ref, w2_ref, b2_ref, w3_ref, b3_ref, out_ref):
    agg = agg_ref[0] + agg_ref[1]
    deg = deg_ref[0, :, 0:1] + deg_ref[1, :, 0:1]
    mean = agg / jnp.maximum(deg, 1.0)
    f32 = jnp.float32
    h = (jnp.dot(z_ref[...], ws_ref[...], preferred_element_type=f32)
         + jnp.dot(mean, wn_ref[...], preferred_element_type=f32)
         + bg_ref[...])
    h = jnp.tanh(jnp.dot(h, w1_ref[...], preferred_element_type=f32)
                 + b1_ref[...])
    h = jnp.tanh(jnp.dot(h, w2_ref[...], preferred_element_type=f32)
                 + b2_ref[...])
    out_ref[...] = (jnp.dot(h, w3_ref[...], preferred_element_type=f32)
                    + b3_ref[...])


BR = 1000


def _tc_mlp(z, agg, deg, W_self, W_neigh, b_gnn, W1, b1, W2, b2, W3, b3):
    full = lambda s: pl.BlockSpec(s, lambda i: (0,) * len(s))
    return pl.pallas_call(
        _tc_body,
        grid=(N // BR,),
        in_specs=[
            pl.BlockSpec((BR, D), lambda i: (i, 0)),
            pl.BlockSpec((NC, BR, D), lambda i: (0, i, 0)),
            pl.BlockSpec((NC, BR, DEGW), lambda i: (0, i, 0)),
            full((D, D)), full((D, D)), full((1, D)),
            full((D, H)), full((1, H)),
            full((H, H)), full((1, H)),
            full((H, D)), full((1, D)),
        ],
        out_specs=pl.BlockSpec((BR, D), lambda i: (i, 0)),
        out_shape=jax.ShapeDtypeStruct((N, D), jnp.float32),
    )(z, agg, deg, W_self, W_neigh, b_gnn.reshape(1, D),
      W1, b1.reshape(1, H), W2, b2.reshape(1, H), W3, b3.reshape(1, D))


def kernel(t, z, edge_index, W_self, W_neigh, b_gnn, W1, b1, W2, b2, W3, b3):
    src = edge_index[0].reshape(NW, NCK, K)
    dst = edge_index[1].reshape(NW, NCK, K)
    zagg = jnp.zeros((NPAD, D), jnp.float32)
    zdeg = jnp.zeros((NPAD, DEGW), jnp.float32)
    ones = jnp.ones((K, DEGW), jnp.float32)
    agg = _sc_aggregate(z, src, dst, zagg)
    deg = _sc_degree(dst, zdeg, ones)
    return _tc_mlp(z, agg, deg, W_self, W_neigh, b_gnn,
                   W1, b1, W2, b2, W3, b3)
